# baseline (speedup 1.0000x reference)
import functools

import jax
import jax.numpy as jnp
from jax import lax
from jax.experimental import pallas as pl
from jax.experimental.pallas import tpu as pltpu
from jax.experimental.pallas import tpu_sc as plsc

TOKENS = 16384
N_EMBD = 2048
N_EXPERTS = 8
TOPK = 2
BLOCK = 512

NUM_CORES = 2
NUM_SUBCORES = 16
LANES = 16
NUM_WORKERS = NUM_CORES * NUM_SUBCORES
TOK_PER_WORKER = TOKENS // NUM_WORKERS


def _scores_block(wt_ref, x_ref, st_ref):
    st = lax.dot_general(
        wt_ref[...], x_ref[...],
        dimension_numbers=(((1,), (1,)), ((), ())),
        preferred_element_type=jnp.float32,
    )
    st_ref[...] = jax.nn.sigmoid(st)


def _tc_scores_t(x, weight):
    n_tokens = x.shape[0]
    return pl.pallas_call(
        _scores_block,
        grid=(n_tokens // BLOCK,),
        in_specs=[
            pl.BlockSpec((N_EXPERTS, N_EMBD), lambda i: (0, 0)),
            pl.BlockSpec((BLOCK, N_EMBD), lambda i: (i, 0)),
        ],
        out_specs=pl.BlockSpec((N_EXPERTS, BLOCK), lambda i: (0, i)),
        out_shape=jax.ShapeDtypeStruct((N_EXPERTS, n_tokens), jnp.float32),
        compiler_params=pltpu.CompilerParams(
            dimension_semantics=("arbitrary",),
        ),
    )(weight.T, x)


@functools.partial(
    pl.kernel,
    mesh=plsc.VectorSubcoreMesh(core_axis_name="c", subcore_axis_name="s"),
    out_type=[
        jax.ShapeDtypeStruct((TOPK, TOKENS), jnp.float32),
        jax.ShapeDtypeStruct((TOPK, TOKENS), jnp.int32),
    ],
    scratch_types=[
        pltpu.VMEM((N_EXPERTS, TOK_PER_WORKER), jnp.float32),
        pltpu.VMEM((TOPK, TOK_PER_WORKER), jnp.float32),
        pltpu.VMEM((TOPK, TOK_PER_WORKER), jnp.int32),
    ],
    compiler_params=pltpu.CompilerParams(needs_layout_passes=False),
)
def _sc_route(st_hbm, wt_hbm, it_hbm, s_v, w_v, i_v):
    wid = lax.axis_index("s") * NUM_CORES + lax.axis_index("c")
    base = wid * TOK_PER_WORKER
    pltpu.sync_copy(st_hbm.at[:, pl.ds(base, TOK_PER_WORKER)], s_v)

    def chunk(c, carry):
        off = c * LANES
        cols = [s_v[e, pl.ds(off, LANES)] for e in range(N_EXPERTS)]
        m1 = cols[0]
        for e in range(1, N_EXPERTS):
            m1 = jnp.maximum(m1, cols[e])
        i1 = jnp.full((LANES,), N_EXPERTS - 1, jnp.int32)
        for e in range(N_EXPERTS - 2, -1, -1):
            i1 = jnp.where(cols[e] == m1, e, i1)
        rest = [jnp.where(i1 == e, -1.0, cols[e]) for e in range(N_EXPERTS)]
        m2 = rest[0]
        for e in range(1, N_EXPERTS):
            m2 = jnp.maximum(m2, rest[e])
        i2 = jnp.full((LANES,), N_EXPERTS - 1, jnp.int32)
        for e in range(N_EXPERTS - 2, -1, -1):
            i2 = jnp.where(rest[e] == m2, e, i2)
        denom = m1 + m2 + 1e-6
        w_v[0, pl.ds(off, LANES)] = m1 / denom
        w_v[1, pl.ds(off, LANES)] = m2 / denom
        i_v[0, pl.ds(off, LANES)] = i1
        i_v[1, pl.ds(off, LANES)] = i2
        return carry

    lax.fori_loop(0, TOK_PER_WORKER // LANES, chunk, 0)

    pltpu.sync_copy(w_v, wt_hbm.at[:, pl.ds(base, TOK_PER_WORKER)])
    pltpu.sync_copy(i_v, it_hbm.at[:, pl.ds(base, TOK_PER_WORKER)])


def kernel(x, weight):
    st = _tc_scores_t(x, weight)
    wt, it = _sc_route(st)
    return (st.T, wt.T, it.T)


# speedup vs baseline: 1.1255x; 1.1255x over previous
import functools

import jax
import jax.numpy as jnp
from jax import lax
from jax.experimental import pallas as pl
from jax.experimental.pallas import tpu as pltpu
from jax.experimental.pallas import tpu_sc as plsc

TOKENS = 16384
N_EMBD = 2048
N_EXPERTS = 8
TOPK = 2
BLOCK = 1024

NUM_CORES = 2
NUM_SUBCORES = 16
LANES = 16
NUM_WORKERS = NUM_CORES * NUM_SUBCORES
TOK_PER_WORKER = TOKENS // NUM_WORKERS


def _scores_block(wt_ref, x_ref, st_ref):
    st = lax.dot_general(
        wt_ref[...], x_ref[...],
        dimension_numbers=(((1,), (1,)), ((), ())),
        preferred_element_type=jnp.float32,
    )
    st_ref[...] = jax.nn.sigmoid(st)


def _tc_scores_t(x, weight):
    n_tokens = x.shape[0]
    return pl.pallas_call(
        _scores_block,
        grid=(n_tokens // BLOCK,),
        in_specs=[
            pl.BlockSpec((N_EXPERTS, N_EMBD), lambda i: (0, 0)),
            pl.BlockSpec((BLOCK, N_EMBD), lambda i: (i, 0)),
        ],
        out_specs=pl.BlockSpec((N_EXPERTS, BLOCK), lambda i: (0, i)),
        out_shape=jax.ShapeDtypeStruct((N_EXPERTS, n_tokens), jnp.float32),
        compiler_params=pltpu.CompilerParams(
            dimension_semantics=("arbitrary",),
        ),
    )(weight.T, x)


@functools.partial(
    pl.kernel,
    mesh=plsc.VectorSubcoreMesh(core_axis_name="c", subcore_axis_name="s"),
    out_type=[
        jax.ShapeDtypeStruct((TOPK, TOKENS), jnp.float32),
        jax.ShapeDtypeStruct((TOPK, TOKENS), jnp.int32),
    ],
    scratch_types=[
        pltpu.VMEM((N_EXPERTS, TOK_PER_WORKER), jnp.float32),
        pltpu.VMEM((TOPK, TOK_PER_WORKER), jnp.float32),
        pltpu.VMEM((TOPK, TOK_PER_WORKER), jnp.int32),
    ],
    compiler_params=pltpu.CompilerParams(needs_layout_passes=False),
)
def _sc_route(st_hbm, wt_hbm, it_hbm, s_v, w_v, i_v):
    wid = lax.axis_index("s") * NUM_CORES + lax.axis_index("c")
    base = wid * TOK_PER_WORKER
    pltpu.sync_copy(st_hbm.at[:, pl.ds(base, TOK_PER_WORKER)], s_v)

    def chunk(c, carry):
        off = c * LANES
        cols = [s_v[e, pl.ds(off, LANES)] for e in range(N_EXPERTS)]
        m1 = cols[0]
        for e in range(1, N_EXPERTS):
            m1 = jnp.maximum(m1, cols[e])
        i1 = jnp.full((LANES,), N_EXPERTS - 1, jnp.int32)
        for e in range(N_EXPERTS - 2, -1, -1):
            i1 = jnp.where(cols[e] == m1, e, i1)
        rest = [jnp.where(i1 == e, -1.0, cols[e]) for e in range(N_EXPERTS)]
        m2 = rest[0]
        for e in range(1, N_EXPERTS):
            m2 = jnp.maximum(m2, rest[e])
        i2 = jnp.full((LANES,), N_EXPERTS - 1, jnp.int32)
        for e in range(N_EXPERTS - 2, -1, -1):
            i2 = jnp.where(rest[e] == m2, e, i2)
        denom = m1 + m2 + 1e-6
        w_v[0, pl.ds(off, LANES)] = m1 / denom
        w_v[1, pl.ds(off, LANES)] = m2 / denom
        i_v[0, pl.ds(off, LANES)] = i1
        i_v[1, pl.ds(off, LANES)] = i2
        return carry

    lax.fori_loop(0, TOK_PER_WORKER // LANES, chunk, 0)

    pltpu.sync_copy(w_v, wt_hbm.at[:, pl.ds(base, TOK_PER_WORKER)])
    pltpu.sync_copy(i_v, it_hbm.at[:, pl.ds(base, TOK_PER_WORKER)])


def kernel(x, weight):
    st = _tc_scores_t(x, weight)
    wt, it = _sc_route(st)
    return (st.T, wt.T, it.T)
